# baseline (speedup 1.0000x reference)
import jax
import jax.numpy as jnp
from jax import lax
from jax.experimental import pallas as pl
from jax.experimental.pallas import tpu as pltpu

_ALPHA = 0.5
_EPS = 1e-05
_N = 16384
_BI = 512
_BJ = 2048


def _cox_kernel(kcol_ref, scol_ref, ccol_ref, krow_ref, srow_ref,
                out_ref, num_ref, den_ref):
    step = pl.program_id(0)
    nsteps = pl.num_programs(0)

    srow = srow_ref[...]
    gamma = jnp.max(srow)
    w_row = jnp.exp(srow - gamma)
    k_row = krow_ref[...]
    k_col = kcol_ref[...]

    ii = step * _BI + lax.broadcasted_iota(jnp.int32, (_BI, _BJ), 0)
    jj0 = lax.broadcasted_iota(jnp.int32, (_BI, _BJ), 1)

    acc = jnp.zeros((_BI, _BJ), jnp.float32)
    for c in range(_N // _BJ):
        kj = k_row[:, c * _BJ:(c + 1) * _BJ]
        wj = w_row[:, c * _BJ:(c + 1) * _BJ]
        jj = jj0 + (c * _BJ)
        mask = (kj > k_col) | ((kj == k_col) & (jj <= ii))
        acc = acc + jnp.where(mask, wj, 0.0)

    p = jnp.sum(acc, axis=1, keepdims=True)
    e = 1.0 - _ALPHA * ccol_ref[...]
    pnum = jnp.sum(e * (gamma + jnp.log(p + _EPS) - scol_ref[...]))
    pden = jnp.sum(e)

    @pl.when(step == 0)
    def _():
        num_ref[0, 0] = 0.0
        den_ref[0, 0] = 0.0

    num_ref[0, 0] += pnum
    den_ref[0, 0] += pden

    @pl.when(step == nsteps - 1)
    def _():
        out_ref[...] = jnp.full((1, 1), num_ref[0, 0] / den_ref[0, 0],
                                jnp.float32)


def kernel(S, c, event_time):
    k = lax.bitcast_convert_type(event_time, jnp.int32)
    kcol = k.reshape(_N, 1)
    krow = k.reshape(1, _N)
    scol = S.reshape(_N, 1)
    srow = S.reshape(1, _N)
    ccol = c.astype(jnp.float32).reshape(_N, 1)

    out = pl.pallas_call(
        _cox_kernel,
        grid=(_N // _BI,),
        in_specs=[
            pl.BlockSpec((_BI, 1), lambda i: (i, 0)),
            pl.BlockSpec((_BI, 1), lambda i: (i, 0)),
            pl.BlockSpec((_BI, 1), lambda i: (i, 0)),
            pl.BlockSpec((1, _N), lambda i: (0, 0)),
            pl.BlockSpec((1, _N), lambda i: (0, 0)),
        ],
        out_specs=pl.BlockSpec((1, 1), lambda i: (0, 0)),
        out_shape=jax.ShapeDtypeStruct((1, 1), jnp.float32),
        scratch_shapes=[
            pltpu.SMEM((1, 1), jnp.float32),
            pltpu.SMEM((1, 1), jnp.float32),
        ],
    )(kcol, scol, ccol, krow, srow)
    return out.reshape(())


# speedup vs baseline: 27.8169x; 27.8169x over previous
import jax
import jax.numpy as jnp
from jax import lax
from jax.experimental import pallas as pl

_ALPHA = 0.5
_EPS = 1e-05
_N = 16384
_R = 128
_L = 128


def _xor_shuffle(x, dist, pbit):
    if dist < _L:
        fwd = jnp.roll(x, dist, axis=1)
        bwd = jnp.roll(x, -dist, axis=1)
    else:
        d = dist // _L
        fwd = jnp.roll(x, d, axis=0)
        bwd = jnp.roll(x, -d, axis=0)
    return jnp.where(pbit, fwd, bwd)


def _cox_kernel(k_ref, s_ref, c_ref, out_ref):
    s = s_ref[...]
    gamma = jnp.max(s)
    w = jnp.exp(s - gamma)
    cbit = c_ref[...] > 0
    e = jnp.where(cbit, _ALPHA, 1.0)
    den = jnp.sum(e)
    b_sum = jnp.sum(e * s)
    pay = jnp.where(cbit, -w, w)
    key = -k_ref[...]

    row = lax.broadcasted_iota(jnp.int32, (_R, _L), 0)
    col = lax.broadcasted_iota(jnp.int32, (_R, _L), 1)
    p = row * _L + col

    size = 2
    while size <= _N:
        dist = size // 2
        while dist >= 1:
            pbit = (p & dist) != 0
            up = (p & size) == 0
            sel_max = up == pbit
            ko = _xor_shuffle(key, dist, pbit)
            po = _xor_shuffle(pay, dist, pbit)
            take = (sel_max & (ko > key)) | (~sel_max & (ko < key))
            key = jnp.where(take, ko, key)
            pay = jnp.where(take, po, pay)
            dist //= 2
        size *= 2

    w_s = jnp.abs(pay)
    e_s = jnp.where(pay < 0.0, _ALPHA, 1.0)
    cs = w_s
    d = 1
    while d < _L:
        cs = cs + jnp.where(col >= d, jnp.roll(cs, d, axis=1), 0.0)
        d *= 2
    row_tot = cs[:, _L - 1:_L]
    tri = jnp.where(lax.broadcasted_iota(jnp.int32, (_R, _R), 0)
                    > lax.broadcasted_iota(jnp.int32, (_R, _R), 1), 1.0, 0.0)
    row_off = jax.lax.dot_general(tri, row_tot, (((1,), (0,)), ((), ())),
                                  preferred_element_type=jnp.float32)
    ptot = cs + row_off
    a_sum = jnp.sum(e_s * jnp.log(ptot + _EPS))
    out_ref[...] = jnp.full((1, 1), (a_sum + gamma * den - b_sum) / den,
                            jnp.float32)


def kernel(S, c, event_time):
    k = lax.bitcast_convert_type(event_time, jnp.int32).reshape(_R, _L)
    s2 = S.reshape(_R, _L)
    c2 = c.astype(jnp.int32).reshape(_R, _L)
    out = pl.pallas_call(
        _cox_kernel,
        out_shape=jax.ShapeDtypeStruct((1, 1), jnp.float32),
    )(k, s2, c2)
    return out.reshape(())


# speedup vs baseline: 32.9942x; 1.1861x over previous
import jax
import jax.numpy as jnp
from jax import lax
from jax.experimental import pallas as pl

_ALPHA = 0.5
_EPS = 1e-05
_N = 16384
_R = 128
_L = 128


def _xor_shuffle(x, dist, pbit):
    if dist < _L:
        fwd = jnp.roll(x, dist, axis=1)
        bwd = jnp.roll(x, -dist, axis=1)
    else:
        d = dist // _L
        fwd = jnp.roll(x, d, axis=0)
        bwd = jnp.roll(x, -d, axis=0)
    return jnp.where(pbit, fwd, bwd)


def _cox_kernel(k_ref, s_ref, c_ref, out_ref):
    s = s_ref[...]
    gamma = jnp.max(s)
    w = jnp.exp(s - gamma)
    cbit = c_ref[...] > 0
    e = jnp.where(cbit, _ALPHA, 1.0)
    den = jnp.sum(e)
    b_sum = jnp.sum(e * s)
    pay = jnp.where(cbit, -w, w)
    key = -k_ref[...]

    row = lax.broadcasted_iota(jnp.int32, (_R, _L), 0)
    col = lax.broadcasted_iota(jnp.int32, (_R, _L), 1)
    p = row * _L + col

    size = 2
    while size <= _N:
        dist = size // 2
        while dist >= 1:
            pbit = (p & dist) != 0
            up = (p & size) == 0
            sel_max = up == pbit
            ko = _xor_shuffle(key, dist, pbit)
            po = _xor_shuffle(pay, dist, pbit)
            knew = jnp.where(sel_max, jnp.maximum(key, ko),
                             jnp.minimum(key, ko))
            pay = jnp.where(knew != key, po, pay)
            key = knew
            dist //= 2
        size *= 2

    w_s = jnp.abs(pay)
    e_s = jnp.where(pay < 0.0, _ALPHA, 1.0)
    cs = w_s
    d = 1
    while d < _L:
        cs = cs + jnp.where(col >= d, jnp.roll(cs, d, axis=1), 0.0)
        d *= 2
    row_tot = cs[:, _L - 1:_L]
    tri = jnp.where(lax.broadcasted_iota(jnp.int32, (_R, _R), 0)
                    > lax.broadcasted_iota(jnp.int32, (_R, _R), 1), 1.0, 0.0)
    row_off = jax.lax.dot_general(tri, row_tot, (((1,), (0,)), ((), ())),
                                  preferred_element_type=jnp.float32)
    ptot = cs + row_off
    a_sum = jnp.sum(e_s * jnp.log(ptot + _EPS))
    out_ref[...] = jnp.full((1, 1), (a_sum + gamma * den - b_sum) / den,
                            jnp.float32)


def kernel(S, c, event_time):
    k = lax.bitcast_convert_type(event_time, jnp.int32).reshape(_R, _L)
    s2 = S.reshape(_R, _L)
    c2 = c.astype(jnp.int32).reshape(_R, _L)
    out = pl.pallas_call(
        _cox_kernel,
        out_shape=jax.ShapeDtypeStruct((1, 1), jnp.float32),
    )(k, s2, c2)
    return out.reshape(())


# speedup vs baseline: 33.4720x; 1.0145x over previous
import jax
import jax.numpy as jnp
from jax import lax
from jax.experimental import pallas as pl

_ALPHA = 0.5
_EPS = 1e-05
_N = 16384
_R = 128
_L = 128


def _xor_shuffle(x, dist, pbit):
    if dist < _L:
        fwd = jnp.roll(x, dist, axis=1)
        bwd = jnp.roll(x, -dist, axis=1)
    else:
        d = dist // _L
        fwd = jnp.roll(x, d, axis=0)
        bwd = jnp.roll(x, -d, axis=0)
    return jnp.where(pbit, fwd, bwd)


def _cox_kernel(k_ref, s_ref, c_ref, out_ref):
    s = s_ref[...]
    gamma = jnp.max(s)
    w = jnp.exp(s - gamma)
    cbit = c_ref[...] > 0
    e = jnp.where(cbit, _ALPHA, 1.0)
    den = jnp.sum(e)
    b_sum = jnp.sum(e * s)
    pay = jnp.where(cbit, -w, w)
    key = -k_ref[...]

    row = lax.broadcasted_iota(jnp.int32, (_R, _L), 0)
    col = lax.broadcasted_iota(jnp.int32, (_R, _L), 1)
    p = row * _L + col

    bits = [(p & (1 << b)) != 0 for b in range(14)]

    size = 2
    while size <= _N:
        dist = size // 2
        while dist >= 1:
            pbit = bits[dist.bit_length() - 1]
            if size <= _N // 2:
                sel_max = bits[size.bit_length() - 1] ^ pbit
            else:
                sel_max = pbit
            ko = _xor_shuffle(key, dist, pbit)
            po = _xor_shuffle(pay, dist, pbit)
            knew = jnp.where(sel_max, jnp.maximum(key, ko),
                             jnp.minimum(key, ko))
            pay = jnp.where(knew != key, po, pay)
            key = knew
            dist //= 2
        size *= 2

    w_s = jnp.abs(pay)
    e_s = jnp.where(pay < 0.0, _ALPHA, 1.0)
    cs = w_s
    d = 1
    while d < _L:
        cs = cs + jnp.where(col >= d, jnp.roll(cs, d, axis=1), 0.0)
        d *= 2
    row_tot = cs[:, _L - 1:_L]
    tri = jnp.where(lax.broadcasted_iota(jnp.int32, (_R, _R), 0)
                    > lax.broadcasted_iota(jnp.int32, (_R, _R), 1), 1.0, 0.0)
    row_off = jax.lax.dot_general(tri, row_tot, (((1,), (0,)), ((), ())),
                                  preferred_element_type=jnp.float32)
    ptot = cs + row_off
    a_sum = jnp.sum(e_s * jnp.log(ptot + _EPS))
    out_ref[...] = jnp.full((1, 1), (a_sum + gamma * den - b_sum) / den,
                            jnp.float32)


def kernel(S, c, event_time):
    k = lax.bitcast_convert_type(event_time, jnp.int32).reshape(_R, _L)
    s2 = S.reshape(_R, _L)
    c2 = c.astype(jnp.int32).reshape(_R, _L)
    out = pl.pallas_call(
        _cox_kernel,
        out_shape=jax.ShapeDtypeStruct((1, 1), jnp.float32),
    )(k, s2, c2)
    return out.reshape(())


# speedup vs baseline: 36.4853x; 1.0900x over previous
import jax
import jax.numpy as jnp
from jax import lax
from jax.experimental import pallas as pl

_ALPHA = 0.5
_EPS = 1e-05
_N = 16384
_R = 128
_L = 128


def _xor_shuffle(x, dist, pbit):
    if dist < _L:
        fwd = jnp.roll(x, dist, axis=1)
        bwd = jnp.roll(x, -dist, axis=1)
        return jnp.where(pbit, fwd, bwd)
    d = dist // _L
    return jnp.concatenate(
        [x[(blk ^ 1) * d:((blk ^ 1) + 1) * d] for blk in range(_R // d)],
        axis=0)


def _cox_kernel(k_ref, s_ref, c_ref, out_ref):
    s = s_ref[...]
    gamma = jnp.max(s)
    w = jnp.exp(s - gamma)
    cbit = c_ref[...] > 0
    e = jnp.where(cbit, _ALPHA, 1.0)
    den = jnp.sum(e)
    b_sum = jnp.sum(e * s)
    pay = jnp.where(cbit, -w, w)
    key = -k_ref[...]

    row = lax.broadcasted_iota(jnp.int32, (_R, _L), 0)
    col = lax.broadcasted_iota(jnp.int32, (_R, _L), 1)
    p = row * _L + col

    bits = [(p & (1 << b)) != 0 for b in range(14)]

    def stage(k, v, size, dist, bts):
        pbit = bts[dist.bit_length() - 1]
        if size <= _N // 2:
            sel_max = bts[size.bit_length() - 1] ^ pbit
        else:
            sel_max = pbit
        ko = _xor_shuffle(k, dist, pbit)
        vo = _xor_shuffle(v, dist, pbit)
        kn = jnp.where(sel_max, jnp.maximum(k, ko), jnp.minimum(k, ko))
        return kn, jnp.where(kn != k, vo, v)

    kc, vc = [], []
    for i in range(_R // 8):
        sl = slice(8 * i, 8 * i + 8)
        k, v = key[sl], pay[sl]
        bts = [b[sl] for b in bits]
        size = 2
        while size <= _L:
            dist = size // 2
            while dist >= 1:
                k, v = stage(k, v, size, dist, bts)
                dist //= 2
            size *= 2
        kc.append(k)
        vc.append(v)
    key, pay = jnp.concatenate(kc, axis=0), jnp.concatenate(vc, axis=0)

    size = 2 * _L
    while size <= _N:
        dist = size // 2
        while dist >= _L:
            key, pay = stage(key, pay, size, dist, bits)
            dist //= 2
        kc, vc = [], []
        for i in range(_R // 8):
            sl = slice(8 * i, 8 * i + 8)
            k, v = key[sl], pay[sl]
            bts = [b[sl] for b in bits]
            d = _L // 2
            while d >= 1:
                k, v = stage(k, v, size, d, bts)
                d //= 2
            kc.append(k)
            vc.append(v)
        key, pay = jnp.concatenate(kc, axis=0), jnp.concatenate(vc, axis=0)
        size *= 2

    w_s = jnp.abs(pay)
    e_s = jnp.where(pay < 0.0, _ALPHA, 1.0)
    cs = w_s
    d = 1
    while d < _L:
        cs = cs + jnp.where(col >= d, jnp.roll(cs, d, axis=1), 0.0)
        d *= 2
    row_tot = cs[:, _L - 1:_L]
    tri = jnp.where(lax.broadcasted_iota(jnp.int32, (_R, _R), 0)
                    > lax.broadcasted_iota(jnp.int32, (_R, _R), 1), 1.0, 0.0)
    row_off = jax.lax.dot_general(tri, row_tot, (((1,), (0,)), ((), ())),
                                  preferred_element_type=jnp.float32)
    ptot = cs + row_off
    a_sum = jnp.sum(e_s * jnp.log(ptot + _EPS))
    out_ref[...] = jnp.full((1, 1), (a_sum + gamma * den - b_sum) / den,
                            jnp.float32)


def kernel(S, c, event_time):
    k = lax.bitcast_convert_type(event_time, jnp.int32).reshape(_R, _L)
    s2 = S.reshape(_R, _L)
    c2 = c.astype(jnp.int32).reshape(_R, _L)
    out = pl.pallas_call(
        _cox_kernel,
        out_shape=jax.ShapeDtypeStruct((1, 1), jnp.float32),
    )(k, s2, c2)
    return out.reshape(())
